# Initial kernel scaffold; baseline (speedup 1.0000x reference)
import jax
import jax.numpy as jnp
from jax.experimental import pallas as pl


def kernel(lengths, ray_weights):
    raise NotImplementedError("write your pallas kernel here")


# speedup vs baseline: 10.8085x; 10.8085x over previous
import functools

import jax
import jax.numpy as jnp
from jax import lax
from jax.experimental import pallas as pl
from jax.experimental.pallas import tpu as pltpu
from jax.experimental.pallas import tpu_sc as plsc

N_RAYS = 131072
N_PTS = 64
N_OUT = 128
EPS = 1e-5
L = 16
NWORKERS = 32
CHUNK = 128
ROWS_PER_W = N_RAYS // NWORKERS
NCHUNKS = ROWS_PER_W // CHUNK


def _hw_sort(v):
    return lax.sort(v, dimension=0)


def _rev(v):
    return lax.rev(v, (0,))


def _bitonic(vs):
    if len(vs) == 1:
        return [_hw_sort(vs[0])]
    h = len(vs) // 2
    lo = [jnp.minimum(vs[i], vs[i + h]) for i in range(h)]
    hi = [jnp.maximum(vs[i], vs[i + h]) for i in range(h)]
    return _bitonic(lo) + _bitonic(hi)


def _merge(a, b):
    return _bitonic(a + [_rev(x) for x in reversed(b)])


def _sort128(vs):
    s = [_hw_sort(v) for v in vs]
    s32 = [_merge([s[0]], [s[1]]), _merge([s[2]], [s[3]]),
           _merge([s[4]], [s[5]]), _merge([s[6]], [s[7]])]
    s64 = [_merge(s32[0], s32[1]), _merge(s32[2], s32[3])]
    return _merge(s64[0], s64[1])


def _body(z_hbm, w_hbm, u_hbm, out_hbm, z_v, w_v, o_v, u_v, cdf_v, hist_v):
    wid = lax.axis_index("s") * 2 + lax.axis_index("c")

    pltpu.sync_copy(u_hbm, u_v)
    u_regs = [u_v[pl.ds(i * L, L)] for i in range(4)]
    iota = lax.iota(jnp.int32, L)
    mask_not_last = iota < (L - 1)
    ones_i = jnp.ones((L,), jnp.int32)
    zeros_i = jnp.zeros((L,), jnp.int32)

    def ray_body(r, carry):
        zr = [z_v[r, pl.ds(i * L, L)] for i in range(4)]
        br = [w_v[r, pl.ds(i * L, L)] + EPS for i in range(4)]

        cs = []
        pref = jnp.float32(0.0)
        for i in range(4):
            cs.append(plsc.cumsum(br[i]) + pref)
            pref = pref + jnp.sum(br[i])
        csum0 = br[0][0]
        b63 = br[3][L - 1]
        denom_total = pref - b63 - csum0
        rcp = jnp.float32(1.0) / jnp.full((L,), denom_total, jnp.float32)
        cdf = [(cs[i] - csum0) * rcp for i in range(4)]

        for i in range(4):
            cdf_v[pl.ds(i * L, L)] = cdf[i]
            hist_v[pl.ds(i * L, L)] = zeros_i

        for i in range(4):
            x = cdf[i] * jnp.float32(63.0)
            ti = x.astype(jnp.int32)
            m = ti + (ti.astype(jnp.float32) < x).astype(jnp.int32)
            m = jnp.minimum(m, 63)
            if i == 3:
                plsc.addupdate_scatter(hist_v, [m], ones_i, mask=mask_not_last)
            else:
                plsc.addupdate_scatter(hist_v, [m], ones_i)

        inds_regs = []
        ip = jnp.int32(0)
        for i in range(4):
            h = hist_v[pl.ds(i * L, L)]
            inds_regs.append(plsc.cumsum(h) + ip)
            ip = ip + jnp.sum(h)

        rsplat = jnp.full((L,), r, jnp.int32)
        samples = []
        for i in range(4):
            inds = inds_regs[i]
            below = inds - 1
            above = jnp.minimum(inds, 62)
            cg0 = plsc.load_gather(cdf_v, [below])
            cg1 = plsc.load_gather(cdf_v, [above])
            zb0 = plsc.load_gather(z_v, [rsplat, below])
            zb1 = plsc.load_gather(z_v, [rsplat, below + 1])
            za0 = plsc.load_gather(z_v, [rsplat, above])
            za1 = plsc.load_gather(z_v, [rsplat, above + 1])
            bg0 = jnp.float32(0.5) * (zb0 + zb1)
            bg1 = jnp.float32(0.5) * (za0 + za1)
            den = cg1 - cg0
            den = jnp.where(den < EPS, jnp.float32(1.0), den)
            t = (u_regs[i] - cg0) / den
            samples.append(bg0 + t * (bg1 - bg0))

        so = _sort128(zr + samples)
        for i in range(8):
            o_v[r, pl.ds(i * L, L)] = so[i]
        return carry

    def chunk_body(c, carry):
        base = wid * ROWS_PER_W + c * CHUNK
        pltpu.sync_copy(z_hbm.at[pl.ds(base, CHUNK)], z_v)
        pltpu.sync_copy(w_hbm.at[pl.ds(base, CHUNK)], w_v)
        lax.fori_loop(0, CHUNK, ray_body, 0, unroll=False)
        pltpu.sync_copy(o_v, out_hbm.at[pl.ds(base, CHUNK)])
        return carry

    lax.fori_loop(0, NCHUNKS, chunk_body, 0, unroll=False)


def kernel(lengths, ray_weights):
    u = jnp.linspace(0.0, 1.0, N_PTS, dtype=jnp.float32)
    mesh = plsc.VectorSubcoreMesh(core_axis_name="c", subcore_axis_name="s")
    run = pl.kernel(
        _body,
        mesh=mesh,
        out_type=jax.ShapeDtypeStruct((N_RAYS, N_OUT), jnp.float32),
        compiler_params=pltpu.CompilerParams(needs_layout_passes=False),
        scratch_types=[
            pltpu.VMEM((CHUNK, N_PTS), jnp.float32),
            pltpu.VMEM((CHUNK, N_PTS), jnp.float32),
            pltpu.VMEM((CHUNK, N_OUT), jnp.float32),
            pltpu.VMEM((N_PTS,), jnp.float32),
            pltpu.VMEM((N_PTS,), jnp.float32),
            pltpu.VMEM((N_PTS,), jnp.int32),
        ],
    )
    return run(lengths, ray_weights, u)
